# baseline (speedup 1.0000x reference)
import functools

import jax
import jax.numpy as jnp
from jax import lax
from jax.experimental import pallas as pl
from jax.experimental.pallas import tpu as pltpu
from jax.experimental.pallas import tpu_sc as plsc

NODES = 10000
FEATS = 128
CLS = 16
NP = 10240
CHUNK = 128
NCORES = 2
NSUB = 16
NTILES = NCORES * NSUB
RPT = NP // NSUB

def _mesh():
    return plsc.VectorSubcoreMesh(
        core_axis_name="c", subcore_axis_name="s",
        num_cores=NCORES, num_subcores=NSUB)


NBUF = 4


CORE_CHUNKS = (68, 92)


def _make_prop(e_pad: int):
    c0, c1 = CORE_CHUNKS
    grp = c0 + c1
    maxc = max(c0, c1)
    assert e_pad == NSUB * grp * CHUNK and c0 % NBUF == 0 and c1 % NBUF == 0

    @functools.partial(
        pl.kernel,
        out_type=jax.ShapeDtypeStruct((NCORES, NP, CLS), jnp.float32),
        mesh=_mesh(),
        scratch_types=[
            pltpu.VMEM((maxc + NBUF, CHUNK), jnp.int32),
            pltpu.VMEM((maxc, CHUNK), jnp.int32),
            pltpu.VMEM((NBUF, CHUNK, CLS), jnp.float32),
            pltpu.VMEM_SHARED((NP, CLS), jnp.float32),
            [pltpu.SemaphoreType.DMA] * NBUF,
        ],
        compiler_params=pltpu.CompilerParams(use_tc_tiling_on_sc=False),
    )
    def prop(z_hbm, src_hbm, dst_hbm, zero_hbm, out_hbm,
             src_v, dst_v, rows_v, acc_sh, sems):
        cid = lax.axis_index("c")
        sid = lax.axis_index("s")
        r0 = sid * RPT
        pltpu.sync_copy(zero_hbm.at[pl.ds(r0, RPT)], acc_sh.at[pl.ds(r0, RPT)])
        nch = jnp.where(cid == 0, c0, c1)
        start = sid * grp + cid * c0
        pltpu.sync_copy(src_hbm.at[pl.ds(start, maxc)],
                        src_v.at[pl.ds(0, maxc)])
        pltpu.sync_copy(dst_hbm.at[pl.ds(start, maxc)], dst_v)
        for k in range(NBUF):
            for i in range(CHUNK // 16):
                src_v[nch + k, pl.ds(i * 16, 16)] = jnp.zeros(
                    (16,), jnp.int32)
        plsc.subcore_barrier()
        for k in range(NBUF):
            pltpu.async_copy(z_hbm.at[src_v.at[k]], rows_v.at[k], sems[k])

        def body(i, carry):
            c = i * NBUF
            for k in range(NBUF):
                pltpu.make_async_copy(
                    z_hbm.at[src_v.at[c + k]], rows_v.at[k], sems[k]).wait()
                pltpu.sync_copy(rows_v.at[k], acc_sh.at[dst_v.at[c + k]],
                                add=True)
                pltpu.async_copy(
                    z_hbm.at[src_v.at[c + k + NBUF]], rows_v.at[k], sems[k])
            return carry

        lax.fori_loop(0, nch // NBUF, body, 0)
        for k in range(NBUF):
            pltpu.make_async_copy(
                z_hbm.at[src_v.at[nch + k]], rows_v.at[k], sems[k]).wait()
        plsc.subcore_barrier()
        pltpu.sync_copy(acc_sh.at[pl.ds(r0, RPT)],
                        out_hbm.at[cid, pl.ds(r0, RPT)])

    return prop


def _make_degree(e_pad: int):
    chunks = e_pad // (NTILES * CHUNK)

    @functools.partial(
        pl.kernel,
        out_type=jax.ShapeDtypeStruct((NCORES, NP), jnp.float32),
        mesh=_mesh(),
        scratch_types=[
            pltpu.VMEM((chunks, CHUNK), jnp.int32),
            pltpu.VMEM((CHUNK,), jnp.float32),
            pltpu.VMEM_SHARED((NP,), jnp.float32),
        ],
        compiler_params=pltpu.CompilerParams(use_tc_tiling_on_sc=False),
    )
    def degree(dst_hbm, zero_hbm, out_hbm, dst_v, ones_v, deg_sh):
        cid = lax.axis_index("c")
        sid = lax.axis_index("s")
        for i in range(CHUNK // 16):
            ones_v[pl.ds(i * 16, 16)] = jnp.ones((16,), jnp.float32)
        r0 = sid * RPT
        pltpu.sync_copy(zero_hbm.at[pl.ds(r0, RPT)], deg_sh.at[pl.ds(r0, RPT)])
        wid = cid * NSUB + sid
        pltpu.sync_copy(dst_hbm.at[pl.ds(wid * chunks, chunks)], dst_v)
        plsc.subcore_barrier()

        def body(j, carry):
            pltpu.sync_copy(ones_v, deg_sh.at[dst_v.at[j]], add=True)
            return carry

        lax.fori_loop(0, chunks, body, 0)
        plsc.subcore_barrier()
        pltpu.sync_copy(deg_sh.at[pl.ds(r0, RPT)],
                        out_hbm.at[cid, pl.ds(r0, RPT)])

    return degree


def _tc_project(x_ref, w_ref, cnt_ref, z_ref):
    y = jnp.dot(x_ref[...], w_ref[...], preferred_element_type=jnp.float32)
    deg = cnt_ref[:, 0:1] + cnt_ref[:, 1:2] + 1.0
    dinv = lax.rsqrt(deg)
    z_ref[0:NODES, :] = y * dinv[0:NODES, :]


RPT32 = NP // NTILES


def _row_deg(c0_v, c1_v, r):
    idx = jnp.zeros((16,), jnp.int32) + r
    c0 = plsc.load_gather(c0_v, [idx])
    c1 = plsc.load_gather(c1_v, [idx])
    return c0 + c1 + 1.0


def _rsqrt16(x):
    i = plsc.bitcast(x, jnp.int32)
    i = 0x5F3759DF - lax.shift_right_arithmetic(i, 1)
    y = plsc.bitcast(i, jnp.float32)
    for _ in range(3):
        y = y * (1.5 - 0.5 * x * y * y)
    return y


@functools.partial(
    pl.kernel,
    out_type=jax.ShapeDtypeStruct((NP, CLS), jnp.float32),
    mesh=_mesh(),
    scratch_types=[
        pltpu.VMEM((RPT32, CLS), jnp.float32),
        pltpu.VMEM((RPT32, CLS), jnp.float32),
        pltpu.VMEM((RPT32, CLS), jnp.float32),
        pltpu.VMEM((RPT32, CLS), jnp.float32),
        pltpu.VMEM((RPT32,), jnp.float32),
        pltpu.VMEM((RPT32,), jnp.float32),
    ],
    compiler_params=pltpu.CompilerParams(use_tc_tiling_on_sc=False,
                                         needs_layout_passes=False),
)
def _sc_mid(u_hbm, z_hbm, cnt_hbm, z2_hbm, u0_v, u1_v, z_v, z2_v, c0_v, c1_v):
    wid = lax.axis_index("c") * NSUB + lax.axis_index("s")
    r0 = wid * RPT32
    pltpu.sync_copy(u_hbm.at[0, pl.ds(r0, RPT32)], u0_v)
    pltpu.sync_copy(u_hbm.at[1, pl.ds(r0, RPT32)], u1_v)
    pltpu.sync_copy(z_hbm.at[pl.ds(r0, RPT32)], z_v)
    pltpu.sync_copy(cnt_hbm.at[0, pl.ds(r0, RPT32)], c0_v)
    pltpu.sync_copy(cnt_hbm.at[1, pl.ds(r0, RPT32)], c1_v)

    def body(r, carry):
        z2_v[r] = (u0_v[r] + u1_v[r] + z_v[r]) / _row_deg(c0_v, c1_v, r)
        return carry

    lax.fori_loop(0, RPT32, body, 0)
    pltpu.sync_copy(z2_v, z2_hbm.at[pl.ds(r0, RPT32)])


@functools.partial(
    pl.kernel,
    out_type=jax.ShapeDtypeStruct((NP, CLS), jnp.float32),
    mesh=_mesh(),
    scratch_types=[
        pltpu.VMEM((RPT32, CLS), jnp.float32),
        pltpu.VMEM((RPT32, CLS), jnp.float32),
        pltpu.VMEM((RPT32, CLS), jnp.float32),
        pltpu.VMEM((RPT32, CLS), jnp.float32),
        pltpu.VMEM((RPT32,), jnp.float32),
        pltpu.VMEM((RPT32,), jnp.float32),
        pltpu.VMEM((16,), jnp.float32),
    ],
    compiler_params=pltpu.CompilerParams(use_tc_tiling_on_sc=False,
                                         needs_layout_passes=False),
)
def _sc_final(v_hbm, z2_hbm, cnt_hbm, b_hbm, o_hbm,
              v0_v, v1_v, z2_v, o_v, c0_v, c1_v, b_v):
    wid = lax.axis_index("c") * NSUB + lax.axis_index("s")
    r0 = wid * RPT32
    pltpu.sync_copy(v_hbm.at[0, pl.ds(r0, RPT32)], v0_v)
    pltpu.sync_copy(v_hbm.at[1, pl.ds(r0, RPT32)], v1_v)
    pltpu.sync_copy(z2_hbm.at[pl.ds(r0, RPT32)], z2_v)
    pltpu.sync_copy(cnt_hbm.at[0, pl.ds(r0, RPT32)], c0_v)
    pltpu.sync_copy(cnt_hbm.at[1, pl.ds(r0, RPT32)], c1_v)
    pltpu.sync_copy(b_hbm, b_v)

    def body(r, carry):
        dinv = _rsqrt16(_row_deg(c0_v, c1_v, r))
        o_v[r] = (v0_v[r] + v1_v[r] + z2_v[r]) * dinv + b_v[...]
        return carry

    lax.fori_loop(0, RPT32, body, 0)
    pltpu.sync_copy(o_v, o_hbm.at[pl.ds(r0, RPT32)])


def kernel(x, edge_index, W, b):
    e = edge_index.shape[1]
    gran = NTILES * CHUNK * NBUF
    e_pad = ((e + gran - 1) // gran) * gran
    chunks = e_pad // (NTILES * CHUNK)
    pad = e_pad - e
    src = jnp.concatenate(
        [edge_index[0], jnp.zeros((pad,), jnp.int32)]
    ).reshape(NTILES * chunks, CHUNK)
    dst = jnp.concatenate(
        [edge_index[1], jnp.full((pad,), NODES, jnp.int32)]
    ).reshape(NTILES * chunks, CHUNK)
    zeros2d = jnp.zeros((NP, CLS), jnp.float32)
    zeros1d = jnp.zeros((NP,), jnp.float32)

    degree = _make_degree(e_pad)
    prop = _make_prop(e_pad)

    counts = degree(dst, zeros1d)

    z = pl.pallas_call(
        _tc_project,
        out_shape=jax.ShapeDtypeStruct((NP, CLS), jnp.float32),
    )(x, W, counts.T)

    u = prop(z, src, dst, zeros2d)
    z2 = _sc_mid(u, z, counts)
    v = prop(z2, src, dst, zeros2d)
    out = _sc_final(v, z2, counts, b)

    return out[:NODES]


# speedup vs baseline: 1.0283x; 1.0283x over previous
import functools

import jax
import jax.numpy as jnp
from jax import lax
from jax.experimental import pallas as pl
from jax.experimental.pallas import tpu as pltpu
from jax.experimental.pallas import tpu_sc as plsc

NODES = 10000
FEATS = 128
CLS = 16
NP = 10240
CHUNK = 128
NCORES = 2
NSUB = 16
NTILES = NCORES * NSUB
RPT = NP // NSUB

def _mesh():
    return plsc.VectorSubcoreMesh(
        core_axis_name="c", subcore_axis_name="s",
        num_cores=NCORES, num_subcores=NSUB)


NBUF = 4


CORE_CHUNKS = (92, 68)


def _make_prop(e_pad: int):
    c0, c1 = CORE_CHUNKS
    grp = c0 + c1
    maxc = max(c0, c1)
    assert e_pad == NSUB * grp * CHUNK and c0 % NBUF == 0 and c1 % NBUF == 0

    @functools.partial(
        pl.kernel,
        out_type=jax.ShapeDtypeStruct((NCORES, NP, CLS), jnp.float32),
        mesh=_mesh(),
        scratch_types=[
            pltpu.VMEM((maxc + NBUF, CHUNK), jnp.int32),
            pltpu.VMEM((maxc, CHUNK), jnp.int32),
            pltpu.VMEM((NBUF, CHUNK, CLS), jnp.float32),
            pltpu.VMEM_SHARED((NP, CLS), jnp.float32),
            [pltpu.SemaphoreType.DMA] * NBUF,
        ],
        compiler_params=pltpu.CompilerParams(use_tc_tiling_on_sc=False),
    )
    def prop(z_hbm, src_hbm, dst_hbm, zero_hbm, out_hbm,
             src_v, dst_v, rows_v, acc_sh, sems):
        cid = lax.axis_index("c")
        sid = lax.axis_index("s")
        r0 = sid * RPT
        pltpu.sync_copy(zero_hbm.at[pl.ds(r0, RPT)], acc_sh.at[pl.ds(r0, RPT)])
        nch = jnp.where(cid == 0, c0, c1)
        start = sid * grp + cid * c0
        pltpu.sync_copy(src_hbm.at[pl.ds(start, maxc)],
                        src_v.at[pl.ds(0, maxc)])
        pltpu.sync_copy(dst_hbm.at[pl.ds(start, maxc)], dst_v)
        for k in range(NBUF):
            for i in range(CHUNK // 16):
                src_v[nch + k, pl.ds(i * 16, 16)] = jnp.zeros(
                    (16,), jnp.int32)
        plsc.subcore_barrier()
        for k in range(NBUF):
            pltpu.async_copy(z_hbm.at[src_v.at[k]], rows_v.at[k], sems[k])

        def body(i, carry):
            c = i * NBUF
            for k in range(NBUF):
                pltpu.make_async_copy(
                    z_hbm.at[src_v.at[c + k]], rows_v.at[k], sems[k]).wait()
                pltpu.sync_copy(rows_v.at[k], acc_sh.at[dst_v.at[c + k]],
                                add=True)
                pltpu.async_copy(
                    z_hbm.at[src_v.at[c + k + NBUF]], rows_v.at[k], sems[k])
            return carry

        lax.fori_loop(0, nch // NBUF, body, 0)
        for k in range(NBUF):
            pltpu.make_async_copy(
                z_hbm.at[src_v.at[nch + k]], rows_v.at[k], sems[k]).wait()
        plsc.subcore_barrier()
        pltpu.sync_copy(acc_sh.at[pl.ds(r0, RPT)],
                        out_hbm.at[cid, pl.ds(r0, RPT)])

    return prop


def _make_degree(e_pad: int):
    chunks = e_pad // (NTILES * CHUNK)

    @functools.partial(
        pl.kernel,
        out_type=jax.ShapeDtypeStruct((NCORES, NP), jnp.float32),
        mesh=_mesh(),
        scratch_types=[
            pltpu.VMEM((chunks, CHUNK), jnp.int32),
            pltpu.VMEM((CHUNK,), jnp.float32),
            pltpu.VMEM_SHARED((NP,), jnp.float32),
        ],
        compiler_params=pltpu.CompilerParams(use_tc_tiling_on_sc=False),
    )
    def degree(dst_hbm, zero_hbm, out_hbm, dst_v, ones_v, deg_sh):
        cid = lax.axis_index("c")
        sid = lax.axis_index("s")
        for i in range(CHUNK // 16):
            ones_v[pl.ds(i * 16, 16)] = jnp.ones((16,), jnp.float32)
        r0 = sid * RPT
        pltpu.sync_copy(zero_hbm.at[pl.ds(r0, RPT)], deg_sh.at[pl.ds(r0, RPT)])
        wid = cid * NSUB + sid
        pltpu.sync_copy(dst_hbm.at[pl.ds(wid * chunks, chunks)], dst_v)
        plsc.subcore_barrier()

        def body(j, carry):
            pltpu.sync_copy(ones_v, deg_sh.at[dst_v.at[j]], add=True)
            return carry

        lax.fori_loop(0, chunks, body, 0)
        plsc.subcore_barrier()
        pltpu.sync_copy(deg_sh.at[pl.ds(r0, RPT)],
                        out_hbm.at[cid, pl.ds(r0, RPT)])

    return degree


def _tc_project(x_ref, w_ref, cnt_ref, z_ref):
    y = jnp.dot(x_ref[...], w_ref[...], preferred_element_type=jnp.float32)
    deg = cnt_ref[:, 0:1] + cnt_ref[:, 1:2] + 1.0
    dinv = lax.rsqrt(deg)
    z_ref[0:NODES, :] = y * dinv[0:NODES, :]


RPT32 = NP // NTILES


def _row_deg(c0_v, c1_v, r):
    idx = jnp.zeros((16,), jnp.int32) + r
    c0 = plsc.load_gather(c0_v, [idx])
    c1 = plsc.load_gather(c1_v, [idx])
    return c0 + c1 + 1.0


def _rsqrt16(x):
    i = plsc.bitcast(x, jnp.int32)
    i = 0x5F3759DF - lax.shift_right_arithmetic(i, 1)
    y = plsc.bitcast(i, jnp.float32)
    for _ in range(3):
        y = y * (1.5 - 0.5 * x * y * y)
    return y


@functools.partial(
    pl.kernel,
    out_type=jax.ShapeDtypeStruct((NP, CLS), jnp.float32),
    mesh=_mesh(),
    scratch_types=[
        pltpu.VMEM((RPT32, CLS), jnp.float32),
        pltpu.VMEM((RPT32, CLS), jnp.float32),
        pltpu.VMEM((RPT32, CLS), jnp.float32),
        pltpu.VMEM((RPT32, CLS), jnp.float32),
        pltpu.VMEM((RPT32,), jnp.float32),
        pltpu.VMEM((RPT32,), jnp.float32),
    ],
    compiler_params=pltpu.CompilerParams(use_tc_tiling_on_sc=False,
                                         needs_layout_passes=False),
)
def _sc_mid(u_hbm, z_hbm, cnt_hbm, z2_hbm, u0_v, u1_v, z_v, z2_v, c0_v, c1_v):
    wid = lax.axis_index("c") * NSUB + lax.axis_index("s")
    r0 = wid * RPT32
    pltpu.sync_copy(u_hbm.at[0, pl.ds(r0, RPT32)], u0_v)
    pltpu.sync_copy(u_hbm.at[1, pl.ds(r0, RPT32)], u1_v)
    pltpu.sync_copy(z_hbm.at[pl.ds(r0, RPT32)], z_v)
    pltpu.sync_copy(cnt_hbm.at[0, pl.ds(r0, RPT32)], c0_v)
    pltpu.sync_copy(cnt_hbm.at[1, pl.ds(r0, RPT32)], c1_v)

    def body(r, carry):
        z2_v[r] = (u0_v[r] + u1_v[r] + z_v[r]) / _row_deg(c0_v, c1_v, r)
        return carry

    lax.fori_loop(0, RPT32, body, 0)
    pltpu.sync_copy(z2_v, z2_hbm.at[pl.ds(r0, RPT32)])


@functools.partial(
    pl.kernel,
    out_type=jax.ShapeDtypeStruct((NP, CLS), jnp.float32),
    mesh=_mesh(),
    scratch_types=[
        pltpu.VMEM((RPT32, CLS), jnp.float32),
        pltpu.VMEM((RPT32, CLS), jnp.float32),
        pltpu.VMEM((RPT32, CLS), jnp.float32),
        pltpu.VMEM((RPT32, CLS), jnp.float32),
        pltpu.VMEM((RPT32,), jnp.float32),
        pltpu.VMEM((RPT32,), jnp.float32),
        pltpu.VMEM((16,), jnp.float32),
    ],
    compiler_params=pltpu.CompilerParams(use_tc_tiling_on_sc=False,
                                         needs_layout_passes=False),
)
def _sc_final(v_hbm, z2_hbm, cnt_hbm, b_hbm, o_hbm,
              v0_v, v1_v, z2_v, o_v, c0_v, c1_v, b_v):
    wid = lax.axis_index("c") * NSUB + lax.axis_index("s")
    r0 = wid * RPT32
    pltpu.sync_copy(v_hbm.at[0, pl.ds(r0, RPT32)], v0_v)
    pltpu.sync_copy(v_hbm.at[1, pl.ds(r0, RPT32)], v1_v)
    pltpu.sync_copy(z2_hbm.at[pl.ds(r0, RPT32)], z2_v)
    pltpu.sync_copy(cnt_hbm.at[0, pl.ds(r0, RPT32)], c0_v)
    pltpu.sync_copy(cnt_hbm.at[1, pl.ds(r0, RPT32)], c1_v)
    pltpu.sync_copy(b_hbm, b_v)

    def body(r, carry):
        dinv = _rsqrt16(_row_deg(c0_v, c1_v, r))
        o_v[r] = (v0_v[r] + v1_v[r] + z2_v[r]) * dinv + b_v[...]
        return carry

    lax.fori_loop(0, RPT32, body, 0)
    pltpu.sync_copy(o_v, o_hbm.at[pl.ds(r0, RPT32)])


def kernel(x, edge_index, W, b):
    e = edge_index.shape[1]
    gran = NTILES * CHUNK * NBUF
    e_pad = ((e + gran - 1) // gran) * gran
    chunks = e_pad // (NTILES * CHUNK)
    pad = e_pad - e
    src = jnp.concatenate(
        [edge_index[0], jnp.zeros((pad,), jnp.int32)]
    ).reshape(NTILES * chunks, CHUNK)
    dst = jnp.concatenate(
        [edge_index[1], jnp.full((pad,), NODES, jnp.int32)]
    ).reshape(NTILES * chunks, CHUNK)
    zeros2d = jnp.zeros((NP, CLS), jnp.float32)
    zeros1d = jnp.zeros((NP,), jnp.float32)

    degree = _make_degree(e_pad)
    prop = _make_prop(e_pad)

    counts = degree(dst, zeros1d)

    z = pl.pallas_call(
        _tc_project,
        out_shape=jax.ShapeDtypeStruct((NP, CLS), jnp.float32),
    )(x, W, counts.T)

    u = prop(z, src, dst, zeros2d)
    z2 = _sc_mid(u, z, counts)
    v = prop(z2, src, dst, zeros2d)
    out = _sc_final(v, z2, counts, b)

    return out[:NODES]


# speedup vs baseline: 1.3264x; 1.2899x over previous
import functools

import jax
import jax.numpy as jnp
from jax import lax
from jax.experimental import pallas as pl
from jax.experimental.pallas import tpu as pltpu
from jax.experimental.pallas import tpu_sc as plsc

NODES = 10000
FEATS = 128
CLS = 16
NP = 10240
CHUNK = 128
NCORES = 2
NSUB = 16
NTILES = NCORES * NSUB
RPT = NP // NSUB

def _mesh():
    return plsc.VectorSubcoreMesh(
        core_axis_name="c", subcore_axis_name="s",
        num_cores=NCORES, num_subcores=NSUB)


NBUF = 4


CORE_CHUNKS = (92, 68)


def _make_prop(e_pad: int):
    c0, c1 = CORE_CHUNKS
    grp = c0 + c1
    maxc = max(c0, c1)
    assert e_pad == NSUB * grp * CHUNK and c0 % NBUF == 0 and c1 % NBUF == 0

    @functools.partial(
        pl.kernel,
        out_type=jax.ShapeDtypeStruct((NCORES, NP, CLS), jnp.float32),
        mesh=_mesh(),
        scratch_types=[
            pltpu.VMEM((maxc + NBUF, CHUNK), jnp.int32),
            pltpu.VMEM((maxc, CHUNK), jnp.int32),
            pltpu.VMEM((NBUF, CHUNK, CLS), jnp.float32),
            pltpu.VMEM_SHARED((NP, CLS), jnp.float32),
            [pltpu.SemaphoreType.DMA] * NBUF,
        ],
        compiler_params=pltpu.CompilerParams(use_tc_tiling_on_sc=False),
    )
    def prop(z_hbm, src_hbm, dst_hbm, out_hbm,
             src_v, dst_v, rows_v, acc_sh, sems):
        cid = lax.axis_index("c")
        sid = lax.axis_index("s")
        r0 = sid * RPT
        for j in range(CHUNK):
            rows_v[0, j, pl.ds(0, 16)] = jnp.zeros((16,), jnp.float32)
        for m in range(RPT // CHUNK):
            pltpu.sync_copy(rows_v.at[0],
                            acc_sh.at[pl.ds(r0 + m * CHUNK, CHUNK)])
        nch = jnp.where(cid == 0, c0, c1)
        start = sid * grp + cid * c0
        pltpu.sync_copy(src_hbm.at[pl.ds(start, maxc)],
                        src_v.at[pl.ds(0, maxc)])
        pltpu.sync_copy(dst_hbm.at[pl.ds(start, maxc)], dst_v)
        for k in range(NBUF):
            for i in range(CHUNK // 16):
                src_v[nch + k, pl.ds(i * 16, 16)] = jnp.zeros(
                    (16,), jnp.int32)
        plsc.subcore_barrier()
        for k in range(NBUF):
            pltpu.async_copy(z_hbm.at[src_v.at[k]], rows_v.at[k], sems[k])

        def body(i, carry):
            c = i * NBUF
            for k in range(NBUF):
                pltpu.make_async_copy(
                    z_hbm.at[src_v.at[c + k]], rows_v.at[k], sems[k]).wait()
                pltpu.sync_copy(rows_v.at[k], acc_sh.at[dst_v.at[c + k]],
                                add=True)
                pltpu.async_copy(
                    z_hbm.at[src_v.at[c + k + NBUF]], rows_v.at[k], sems[k])
            return carry

        lax.fori_loop(0, nch // NBUF, body, 0)
        for k in range(NBUF):
            pltpu.make_async_copy(
                z_hbm.at[src_v.at[nch + k]], rows_v.at[k], sems[k]).wait()
        plsc.subcore_barrier()
        pltpu.sync_copy(acc_sh.at[pl.ds(r0, RPT)],
                        out_hbm.at[cid, pl.ds(r0, RPT)])

    return prop


def _make_degree(e_pad: int):
    chunks = e_pad // (NTILES * CHUNK)

    @functools.partial(
        pl.kernel,
        out_type=jax.ShapeDtypeStruct((NCORES, NP), jnp.float32),
        mesh=_mesh(),
        scratch_types=[
            pltpu.VMEM((chunks, CHUNK), jnp.int32),
            pltpu.VMEM((CHUNK,), jnp.float32),
            pltpu.VMEM((CHUNK,), jnp.float32),
            pltpu.VMEM_SHARED((NP,), jnp.float32),
        ],
        compiler_params=pltpu.CompilerParams(use_tc_tiling_on_sc=False),
    )
    def degree(dst_hbm, out_hbm, dst_v, zeros_v, ones_v, deg_sh):
        cid = lax.axis_index("c")
        sid = lax.axis_index("s")
        for i in range(CHUNK // 16):
            ones_v[pl.ds(i * 16, 16)] = jnp.ones((16,), jnp.float32)
            zeros_v[pl.ds(i * 16, 16)] = jnp.zeros((16,), jnp.float32)
        r0 = sid * RPT
        for m in range(RPT // CHUNK):
            pltpu.sync_copy(zeros_v, deg_sh.at[pl.ds(r0 + m * CHUNK, CHUNK)])
        wid = cid * NSUB + sid
        pltpu.sync_copy(dst_hbm.at[pl.ds(wid * chunks, chunks)], dst_v)
        plsc.subcore_barrier()

        def body(j, carry):
            pltpu.sync_copy(ones_v, deg_sh.at[dst_v.at[j]], add=True)
            return carry

        lax.fori_loop(0, chunks, body, 0)
        plsc.subcore_barrier()
        pltpu.sync_copy(deg_sh.at[pl.ds(r0, RPT)],
                        out_hbm.at[cid, pl.ds(r0, RPT)])

    return degree


def _tc_project(x_ref, w_ref, cnt_ref, z_ref):
    y = jnp.dot(x_ref[...], w_ref[...], preferred_element_type=jnp.float32)
    deg = cnt_ref[:, 0:1] + cnt_ref[:, 1:2] + 1.0
    dinv = lax.rsqrt(deg)
    z_ref[0:NODES, :] = y * dinv[0:NODES, :]
    z_ref[NODES:NP, :] = jnp.zeros((NP - NODES, CLS), jnp.float32)


RPT32 = NP // NTILES


def _row_deg(c0_v, c1_v, r):
    idx = jnp.zeros((16,), jnp.int32) + r
    c0 = plsc.load_gather(c0_v, [idx])
    c1 = plsc.load_gather(c1_v, [idx])
    return c0 + c1 + 1.0


def _rsqrt16(x):
    i = plsc.bitcast(x, jnp.int32)
    i = 0x5F3759DF - lax.shift_right_arithmetic(i, 1)
    y = plsc.bitcast(i, jnp.float32)
    for _ in range(3):
        y = y * (1.5 - 0.5 * x * y * y)
    return y


@functools.partial(
    pl.kernel,
    out_type=jax.ShapeDtypeStruct((NP, CLS), jnp.float32),
    mesh=_mesh(),
    scratch_types=[
        pltpu.VMEM((RPT32, CLS), jnp.float32),
        pltpu.VMEM((RPT32, CLS), jnp.float32),
        pltpu.VMEM((RPT32, CLS), jnp.float32),
        pltpu.VMEM((RPT32, CLS), jnp.float32),
        pltpu.VMEM((RPT32,), jnp.float32),
        pltpu.VMEM((RPT32,), jnp.float32),
    ],
    compiler_params=pltpu.CompilerParams(use_tc_tiling_on_sc=False,
                                         needs_layout_passes=False),
)
def _sc_mid(u_hbm, z_hbm, cnt_hbm, z2_hbm, u0_v, u1_v, z_v, z2_v, c0_v, c1_v):
    wid = lax.axis_index("c") * NSUB + lax.axis_index("s")
    r0 = wid * RPT32
    pltpu.sync_copy(u_hbm.at[0, pl.ds(r0, RPT32)], u0_v)
    pltpu.sync_copy(u_hbm.at[1, pl.ds(r0, RPT32)], u1_v)
    pltpu.sync_copy(z_hbm.at[pl.ds(r0, RPT32)], z_v)
    pltpu.sync_copy(cnt_hbm.at[0, pl.ds(r0, RPT32)], c0_v)
    pltpu.sync_copy(cnt_hbm.at[1, pl.ds(r0, RPT32)], c1_v)

    def body(r, carry):
        z2_v[r] = (u0_v[r] + u1_v[r] + z_v[r]) / _row_deg(c0_v, c1_v, r)
        return carry

    lax.fori_loop(0, RPT32, body, 0)
    pltpu.sync_copy(z2_v, z2_hbm.at[pl.ds(r0, RPT32)])


@functools.partial(
    pl.kernel,
    out_type=jax.ShapeDtypeStruct((NP, CLS), jnp.float32),
    mesh=_mesh(),
    scratch_types=[
        pltpu.VMEM((RPT32, CLS), jnp.float32),
        pltpu.VMEM((RPT32, CLS), jnp.float32),
        pltpu.VMEM((RPT32, CLS), jnp.float32),
        pltpu.VMEM((RPT32, CLS), jnp.float32),
        pltpu.VMEM((RPT32,), jnp.float32),
        pltpu.VMEM((RPT32,), jnp.float32),
        pltpu.VMEM((16,), jnp.float32),
    ],
    compiler_params=pltpu.CompilerParams(use_tc_tiling_on_sc=False,
                                         needs_layout_passes=False),
)
def _sc_final(v_hbm, z2_hbm, cnt_hbm, b_hbm, o_hbm,
              v0_v, v1_v, z2_v, o_v, c0_v, c1_v, b_v):
    wid = lax.axis_index("c") * NSUB + lax.axis_index("s")
    r0 = wid * RPT32
    pltpu.sync_copy(v_hbm.at[0, pl.ds(r0, RPT32)], v0_v)
    pltpu.sync_copy(v_hbm.at[1, pl.ds(r0, RPT32)], v1_v)
    pltpu.sync_copy(z2_hbm.at[pl.ds(r0, RPT32)], z2_v)
    pltpu.sync_copy(cnt_hbm.at[0, pl.ds(r0, RPT32)], c0_v)
    pltpu.sync_copy(cnt_hbm.at[1, pl.ds(r0, RPT32)], c1_v)
    pltpu.sync_copy(b_hbm, b_v)

    def body(r, carry):
        dinv = _rsqrt16(_row_deg(c0_v, c1_v, r))
        o_v[r] = (v0_v[r] + v1_v[r] + z2_v[r]) * dinv + b_v[...]
        return carry

    lax.fori_loop(0, RPT32, body, 0)
    pltpu.sync_copy(o_v, o_hbm.at[pl.ds(r0, RPT32)])


def kernel(x, edge_index, W, b):
    e = edge_index.shape[1]
    gran = NTILES * CHUNK * NBUF
    e_pad = ((e + gran - 1) // gran) * gran
    chunks = e_pad // (NTILES * CHUNK)
    pad = e_pad - e
    ei_p = jnp.pad(edge_index, ((0, 0), (0, pad)), constant_values=NODES)
    src = ei_p[0].reshape(NTILES * chunks, CHUNK)
    dst = ei_p[1].reshape(NTILES * chunks, CHUNK)

    degree = _make_degree(e_pad)
    prop = _make_prop(e_pad)

    counts = degree(dst)

    z = pl.pallas_call(
        _tc_project,
        out_shape=jax.ShapeDtypeStruct((NP, CLS), jnp.float32),
    )(x, W, counts.T)

    u = prop(z, src, dst)
    z2 = _sc_mid(u, z, counts)
    v = prop(z2, src, dst)
    out = _sc_final(v, z2, counts, b)

    return out[:NODES]
